# baseline (speedup 1.0000x reference)
import functools

import jax
import jax.numpy as jnp
from jax import lax
from jax.experimental import pallas as pl
from jax.experimental.pallas import tpu as pltpu
from jax.experimental.pallas import tpu_sc as plsc

N = 10000
F = 128
EMB = 128
C = 32
NS1 = 10
NS2 = 10
B = 1024

NW = 32
NODES_PER_W = 320
CHUNK = 64
NCHUNKS = NODES_PER_W // CHUNK
SEEDS_PER_W = B // NW

_mesh = plsc.VectorSubcoreMesh(core_axis_name="c", subcore_axis_name="s")


def _worker_id():
    return lax.axis_index("s") * 2 + lax.axis_index("c")


def _mm_body(x_ref, w_ref, o_ref):
    o_ref[...] = jnp.dot(x_ref[...], w_ref[...],
                         preferred_element_type=jnp.float32)


def _tc_matmul(x, w):
    m = x.shape[0]
    blk = 10000
    return pl.pallas_call(
        _mm_body,
        grid=(m // blk,),
        in_specs=[
            pl.BlockSpec((blk, F), lambda i: (i, 0)),
            pl.BlockSpec((F, EMB), lambda i: (0, 0)),
        ],
        out_specs=pl.BlockSpec((blk, EMB), lambda i: (i, 0)),
        out_shape=jax.ShapeDtypeStruct((m, EMB), jnp.float32),
    )(x, w)


_L1_CHUNKS = ((0, 128), (128, 128), (256, 64))


@functools.partial(
    pl.kernel,
    out_type=jax.ShapeDtypeStruct((N, EMB), jnp.float32),
    mesh=_mesh,
    scratch_types=[
        pltpu.VMEM((NS1 * len(_L1_CHUNKS), 128), jnp.int32),
        pltpu.VMEM((NODES_PER_W, EMB), jnp.float32),
        pltpu.SemaphoreType.DMA((len(_L1_CHUNKS),)),
        pltpu.SemaphoreType.DMA((len(_L1_CHUNKS),)),
        pltpu.SemaphoreType.DMA,
    ],
    compiler_params=pltpu.CompilerParams(use_tc_tiling_on_sc=True),
)
def _sc_layer1(y_hbm, adjt_hbm, h1_hbm, idx_v, acc_v, sem_st, sem_ad,
               sem_out):
    base = jnp.minimum(_worker_id() * NODES_PER_W, N - NODES_PER_W)
    base = pl.multiple_of(base, 16)
    nc = len(_L1_CHUNKS)
    stage = []
    for ci, (c0, csz) in enumerate(_L1_CHUNKS):
        cps = [pltpu.async_copy(
                   adjt_hbm.at[pl.ds(d * N + base + c0, csz)],
                   idx_v.at[d * nc + ci, pl.ds(0, csz)], sem_st.at[ci])
               for d in range(NS1)]
        cps.append(pltpu.async_copy(y_hbm.at[pl.ds(base + c0, csz)],
                                    acc_v.at[pl.ds(c0, csz)], sem_st.at[ci]))
        stage.append(cps)
    adds = []
    for ci, (c0, csz) in enumerate(_L1_CHUNKS):
        for cp in stage[ci]:
            cp.wait()
        adds.append([pltpu.async_copy(
                         y_hbm.at[idx_v.at[d * nc + ci, pl.ds(0, csz)]],
                         acc_v.at[pl.ds(c0, csz)], sem_ad.at[ci], add=True)
                     for d in range(NS1)])
    outs = []
    for ci, (c0, csz) in enumerate(_L1_CHUNKS):
        for cp in adds[ci]:
            cp.wait()

        def _relu_row(i, _, _c0=c0):
            for v in range(EMB // 16):
                x = acc_v[i, pl.ds(v * 16, 16)]
                acc_v[i, pl.ds(v * 16, 16)] = jnp.maximum(x, 0.0)
            return 0

        lax.fori_loop(c0, c0 + csz, _relu_row, 0)
        outs.append(pltpu.async_copy(acc_v.at[pl.ds(c0, csz)],
                                     h1_hbm.at[pl.ds(base + c0, csz)],
                                     sem_out))
    for cp in outs:
        cp.wait()


@functools.partial(
    pl.kernel,
    out_type=jax.ShapeDtypeStruct((B, EMB), jnp.float32),
    mesh=_mesh,
    scratch_types=[
        pltpu.VMEM((SEEDS_PER_W,), jnp.int32),
        pltpu.VMEM((NS2, SEEDS_PER_W), jnp.int32),
        pltpu.VMEM((NS2, SEEDS_PER_W), jnp.int32),
        pltpu.VMEM((SEEDS_PER_W, EMB), jnp.float32),
        pltpu.SemaphoreType.DMA((NS2,)),
        pltpu.SemaphoreType.DMA,
        pltpu.SemaphoreType.DMA,
    ],
    compiler_params=pltpu.CompilerParams(use_tc_tiling_on_sc=True),
)
def _sc_layer2(h1_hbm, nodes_hbm, adjt_hbm, out_hbm,
               nodes_v, off_v, idxs_v, acc_v, sem_g, sem_self, sem_ad):
    base = _worker_id() * SEEDS_PER_W
    pltpu.sync_copy(nodes_hbm.at[pl.ds(base, SEEDS_PER_W)], nodes_v)
    self_cp = pltpu.async_copy(h1_hbm.at[nodes_v], acc_v, sem_self)
    for h in range(SEEDS_PER_W // 16):
        nv = nodes_v[pl.ds(h * 16, 16)]
        for d in range(NS2):
            off_v[d, pl.ds(h * 16, 16)] = nv + (d * N)
    gcps = [pltpu.async_copy(adjt_hbm.at[off_v.at[d]], idxs_v.at[d],
                             sem_g.at[d])
            for d in range(NS2)]
    self_cp.wait()
    adds = []
    for d in range(NS2):
        gcps[d].wait()
        adds.append(pltpu.async_copy(h1_hbm.at[idxs_v.at[d]], acc_v,
                                     sem_ad, add=True))
    for cp in adds:
        cp.wait()
    pltpu.sync_copy(acc_v, out_hbm.at[pl.ds(base, SEEDS_PER_W)])


def _head_body(x_ref, w2_ref, wo_ref, o_ref):
    h2 = jnp.maximum(
        jnp.dot(x_ref[...], w2_ref[...], preferred_element_type=jnp.float32),
        0.0)
    logits = jnp.dot(h2, wo_ref[...], preferred_element_type=jnp.float32)
    o_ref[...] = jax.nn.sigmoid(logits)


def _tc_head(s2, w2t, wot):
    return pl.pallas_call(
        _head_body,
        out_shape=jax.ShapeDtypeStruct((B, C), jnp.float32),
    )(s2, w2t, wot)


def kernel(nodes, features, adj, W1, W2, Wout):
    adjt = adj[:, :NS1].astype(jnp.int32).T.reshape(-1)
    w1t = (W1 * (1.0 / (NS1 + 1))).T
    w2t = (W2 * (1.0 / (NS2 + 1))).T
    wot = Wout.T

    y = _tc_matmul(features, w1t)
    h1 = _sc_layer1(y, adjt)
    s2 = _sc_layer2(h1, nodes.astype(jnp.int32), adjt)
    return _tc_head(s2, w2t, wot)


# speedup vs baseline: 1.0104x; 1.0104x over previous
import functools

import jax
import jax.numpy as jnp
from jax import lax
from jax.experimental import pallas as pl
from jax.experimental.pallas import tpu as pltpu
from jax.experimental.pallas import tpu_sc as plsc

N = 10000
F = 128
EMB = 128
C = 32
NS1 = 10
NS2 = 10
B = 1024

NW = 32
NODES_PER_W = 320
SEEDS_PER_W = B // NW

_mesh = plsc.VectorSubcoreMesh(core_axis_name="c", subcore_axis_name="s")


def _worker_id():
    return lax.axis_index("s") * 2 + lax.axis_index("c")


def _mm_body(x_ref, w_ref, o_ref):
    o_ref[...] = jnp.dot(x_ref[...], w_ref[...],
                         preferred_element_type=jnp.float32)


def _tc_matmul(x, w):
    m = x.shape[0]
    blk = 5000
    return pl.pallas_call(
        _mm_body,
        grid=(m // blk,),
        in_specs=[
            pl.BlockSpec((blk, F), lambda i: (i, 0)),
            pl.BlockSpec((F, EMB), lambda i: (0, 0)),
        ],
        out_specs=pl.BlockSpec((blk, EMB), lambda i: (i, 0)),
        out_shape=jax.ShapeDtypeStruct((m, EMB), jnp.float32),
    )(x, w)


_L1_CHUNKS = ((0, 128), (128, 128), (256, 64))


@functools.partial(
    pl.kernel,
    out_type=jax.ShapeDtypeStruct((N, EMB), jnp.float32),
    mesh=_mesh,
    scratch_types=[
        pltpu.VMEM((NS1 * len(_L1_CHUNKS), 128), jnp.int32),
        pltpu.VMEM((NODES_PER_W, EMB), jnp.float32),
        pltpu.SemaphoreType.DMA((len(_L1_CHUNKS),)),
        pltpu.SemaphoreType.DMA((len(_L1_CHUNKS),)),
        pltpu.SemaphoreType.DMA,
    ],
    compiler_params=pltpu.CompilerParams(use_tc_tiling_on_sc=True),
)
def _sc_layer1(y_hbm, adjt_hbm, h1_hbm, idx_v, acc_v, sem_st, sem_ad,
               sem_out):
    base = jnp.minimum(_worker_id() * NODES_PER_W, N - NODES_PER_W)
    base = pl.multiple_of(base, 16)
    nc = len(_L1_CHUNKS)
    stage = []
    for ci, (c0, csz) in enumerate(_L1_CHUNKS):
        cps = [pltpu.async_copy(
                   adjt_hbm.at[pl.ds(d * N + base + c0, csz)],
                   idx_v.at[d * nc + ci, pl.ds(0, csz)], sem_st.at[ci])
               for d in range(NS1)]
        cps.append(pltpu.async_copy(y_hbm.at[pl.ds(base + c0, csz)],
                                    acc_v.at[pl.ds(c0, csz)], sem_st.at[ci]))
        stage.append(cps)
    adds = []
    for ci, (c0, csz) in enumerate(_L1_CHUNKS):
        for cp in stage[ci]:
            cp.wait()
        adds.append([pltpu.async_copy(
                         y_hbm.at[idx_v.at[d * nc + ci, pl.ds(0, csz)]],
                         acc_v.at[pl.ds(c0, csz)], sem_ad.at[ci], add=True)
                     for d in range(NS1)])
    outs = []
    for ci, (c0, csz) in enumerate(_L1_CHUNKS):
        for cp in adds[ci]:
            cp.wait()

        def _relu_row(i, _, _c0=c0):
            for v in range(EMB // 16):
                x = acc_v[i, pl.ds(v * 16, 16)]
                acc_v[i, pl.ds(v * 16, 16)] = jnp.maximum(x, 0.0)
            return 0

        lax.fori_loop(c0, c0 + csz, _relu_row, 0)
        outs.append(pltpu.async_copy(acc_v.at[pl.ds(c0, csz)],
                                     h1_hbm.at[pl.ds(base + c0, csz)],
                                     sem_out))
    for cp in outs:
        cp.wait()


@functools.partial(
    pl.kernel,
    out_type=jax.ShapeDtypeStruct((B, EMB), jnp.float32),
    mesh=_mesh,
    scratch_types=[
        pltpu.VMEM((SEEDS_PER_W,), jnp.int32),
        pltpu.VMEM((NS2, SEEDS_PER_W), jnp.int32),
        pltpu.VMEM((NS2, SEEDS_PER_W), jnp.int32),
        pltpu.VMEM((SEEDS_PER_W, EMB), jnp.float32),
        pltpu.SemaphoreType.DMA((NS2,)),
        pltpu.SemaphoreType.DMA,
        pltpu.SemaphoreType.DMA,
    ],
    compiler_params=pltpu.CompilerParams(use_tc_tiling_on_sc=True),
)
def _sc_layer2(h1_hbm, nodes_hbm, adjt_hbm, out_hbm,
               nodes_v, off_v, idxs_v, acc_v, sem_g, sem_self, sem_ad):
    base = _worker_id() * SEEDS_PER_W
    pltpu.sync_copy(nodes_hbm.at[pl.ds(base, SEEDS_PER_W)], nodes_v)
    self_cp = pltpu.async_copy(h1_hbm.at[nodes_v], acc_v, sem_self)
    for h in range(SEEDS_PER_W // 16):
        nv = nodes_v[pl.ds(h * 16, 16)]
        for d in range(NS2):
            off_v[d, pl.ds(h * 16, 16)] = nv + (d * N)
    gcps = [pltpu.async_copy(adjt_hbm.at[off_v.at[d]], idxs_v.at[d],
                             sem_g.at[d])
            for d in range(NS2)]
    self_cp.wait()
    adds = []
    for d in range(NS2):
        gcps[d].wait()
        adds.append(pltpu.async_copy(h1_hbm.at[idxs_v.at[d]], acc_v,
                                     sem_ad, add=True))
    for cp in adds:
        cp.wait()
    pltpu.sync_copy(acc_v, out_hbm.at[pl.ds(base, SEEDS_PER_W)])


def _head_body(x_ref, w2_ref, wo_ref, o_ref):
    h2 = jnp.maximum(
        jnp.dot(x_ref[...], w2_ref[...], preferred_element_type=jnp.float32),
        0.0)
    logits = jnp.dot(h2, wo_ref[...], preferred_element_type=jnp.float32)
    o_ref[...] = jax.nn.sigmoid(logits)


def _tc_head(s2, w2t, wot):
    return pl.pallas_call(
        _head_body,
        out_shape=jax.ShapeDtypeStruct((B, C), jnp.float32),
    )(s2, w2t, wot)


def kernel(nodes, features, adj, W1, W2, Wout):
    adjt = adj[:, :NS1].astype(jnp.int32).T.reshape(-1)
    w1t = (W1 * (1.0 / (NS1 + 1))).T
    w2t = (W2 * (1.0 / (NS2 + 1))).T
    wot = Wout.T

    y = _tc_matmul(features, w1t)
    h1 = _sc_layer1(y, adjt)
    s2 = _sc_layer2(h1, nodes.astype(jnp.int32), adjt)
    return _tc_head(s2, w2t, wot)


# speedup vs baseline: 1.0227x; 1.0122x over previous
import functools

import jax
import jax.numpy as jnp
from jax import lax
from jax.experimental import pallas as pl
from jax.experimental.pallas import tpu as pltpu
from jax.experimental.pallas import tpu_sc as plsc

N = 10000
F = 128
EMB = 128
C = 32
NS1 = 10
NS2 = 10
B = 1024

NW = 32
NODES_PER_W = 320
SEEDS_PER_W = B // NW

_mesh = plsc.VectorSubcoreMesh(core_axis_name="c", subcore_axis_name="s")


def _worker_id():
    return lax.axis_index("s") * 2 + lax.axis_index("c")


def _mm_body(x_ref, w_ref, o_ref):
    o_ref[...] = jnp.dot(x_ref[...], w_ref[...],
                         preferred_element_type=jnp.float32)


def _tc_matmul(x, w):
    m = x.shape[0]
    blk = 5000
    return pl.pallas_call(
        _mm_body,
        grid=(m // blk,),
        in_specs=[
            pl.BlockSpec((blk, F), lambda i: (i, 0)),
            pl.BlockSpec((F, EMB), lambda i: (0, 0)),
        ],
        out_specs=pl.BlockSpec((blk, EMB), lambda i: (i, 0)),
        out_shape=jax.ShapeDtypeStruct((m, EMB), jnp.float32),
    )(x, w)


_L1_CHUNKS = ((0, 128), (128, 128), (256, 64))


@functools.partial(
    pl.kernel,
    out_type=(jax.ShapeDtypeStruct((N, EMB), jnp.float32),
              jax.ShapeDtypeStruct((NW * NS2 * SEEDS_PER_W,), jnp.int32)),
    mesh=_mesh,
    scratch_types=[
        pltpu.VMEM((NS1 * len(_L1_CHUNKS), 128), jnp.int32),
        pltpu.VMEM((NODES_PER_W, EMB), jnp.float32),
        pltpu.VMEM((SEEDS_PER_W,), jnp.int32),
        pltpu.VMEM((NS2, SEEDS_PER_W), jnp.int32),
        pltpu.VMEM((NS2 * SEEDS_PER_W,), jnp.int32),
        pltpu.SemaphoreType.DMA((len(_L1_CHUNKS),)),
        pltpu.SemaphoreType.DMA((len(_L1_CHUNKS),)),
        pltpu.SemaphoreType.DMA,
        pltpu.SemaphoreType.DMA,
    ],
    compiler_params=pltpu.CompilerParams(use_tc_tiling_on_sc=True),
)
def _sc_layer1(y_hbm, adjt_hbm, nodes_hbm, h1_hbm, idx2_hbm,
               idx_v, acc_v, nodes_v, off_v, idx2_v,
               sem_st, sem_ad, sem_out, sem_g):
    wid = _worker_id()
    base = jnp.minimum(wid * NODES_PER_W, N - NODES_PER_W)
    base = pl.multiple_of(base, 16)
    nc = len(_L1_CHUNKS)
    stage = []
    for ci, (c0, csz) in enumerate(_L1_CHUNKS):
        cps = [pltpu.async_copy(
                   adjt_hbm.at[pl.ds(d * N + base + c0, csz)],
                   idx_v.at[d * nc + ci, pl.ds(0, csz)], sem_st.at[ci])
               for d in range(NS1)]
        cps.append(pltpu.async_copy(y_hbm.at[pl.ds(base + c0, csz)],
                                    acc_v.at[pl.ds(c0, csz)], sem_st.at[ci]))
        stage.append(cps)
    sbase = wid * (NS2 * SEEDS_PER_W)
    pltpu.sync_copy(nodes_hbm.at[pl.ds(wid * SEEDS_PER_W, SEEDS_PER_W)],
                    nodes_v)
    for h in range(SEEDS_PER_W // 16):
        nv = nodes_v[pl.ds(h * 16, 16)]
        for d in range(NS2):
            off_v[d, pl.ds(h * 16, 16)] = nv + (d * N)
    gcps = [pltpu.async_copy(adjt_hbm.at[off_v.at[d]],
                             idx2_v.at[pl.ds(d * SEEDS_PER_W, SEEDS_PER_W)],
                             sem_g)
            for d in range(NS2)]
    adds = []
    for ci, (c0, csz) in enumerate(_L1_CHUNKS):
        for cp in stage[ci]:
            cp.wait()
        adds.append([pltpu.async_copy(
                         y_hbm.at[idx_v.at[d * nc + ci, pl.ds(0, csz)]],
                         acc_v.at[pl.ds(c0, csz)], sem_ad.at[ci], add=True)
                     for d in range(NS1)])
    outs = []
    for ci, (c0, csz) in enumerate(_L1_CHUNKS):
        for cp in adds[ci]:
            cp.wait()

        def _relu_row(i, _, _c0=c0):
            for v in range(EMB // 16):
                x = acc_v[i, pl.ds(v * 16, 16)]
                acc_v[i, pl.ds(v * 16, 16)] = jnp.maximum(x, 0.0)
            return 0

        lax.fori_loop(c0, c0 + csz, _relu_row, 0)
        outs.append(pltpu.async_copy(acc_v.at[pl.ds(c0, csz)],
                                     h1_hbm.at[pl.ds(base + c0, csz)],
                                     sem_out))
    for cp in gcps:
        cp.wait()
    outs.append(pltpu.async_copy(idx2_v, idx2_hbm.at[pl.ds(sbase,
                                                           NS2 * SEEDS_PER_W)],
                                 sem_out))
    for cp in outs:
        cp.wait()


@functools.partial(
    pl.kernel,
    out_type=jax.ShapeDtypeStruct((B, EMB), jnp.float32),
    mesh=_mesh,
    scratch_types=[
        pltpu.VMEM((SEEDS_PER_W,), jnp.int32),
        pltpu.VMEM((NS2 * SEEDS_PER_W,), jnp.int32),
        pltpu.VMEM((SEEDS_PER_W, EMB), jnp.float32),
        pltpu.SemaphoreType.DMA,
        pltpu.SemaphoreType.DMA,
    ],
    compiler_params=pltpu.CompilerParams(use_tc_tiling_on_sc=True),
)
def _sc_layer2(h1_hbm, nodes_hbm, idx2_hbm, out_hbm,
               nodes_v, idxs_v, acc_v, sem_st, sem_ad):
    wid = _worker_id()
    base = wid * SEEDS_PER_W
    icp = pltpu.async_copy(
        idx2_hbm.at[pl.ds(wid * (NS2 * SEEDS_PER_W), NS2 * SEEDS_PER_W)],
        idxs_v, sem_st)
    pltpu.sync_copy(nodes_hbm.at[pl.ds(base, SEEDS_PER_W)], nodes_v)
    self_cp = pltpu.async_copy(h1_hbm.at[nodes_v], acc_v, sem_st)
    icp.wait()
    self_cp.wait()
    adds = [pltpu.async_copy(
                h1_hbm.at[idxs_v.at[pl.ds(d * SEEDS_PER_W, SEEDS_PER_W)]],
                acc_v, sem_ad, add=True)
            for d in range(NS2)]
    for cp in adds:
        cp.wait()
    pltpu.sync_copy(acc_v, out_hbm.at[pl.ds(base, SEEDS_PER_W)])


def _head_body(x_ref, w2_ref, wo_ref, o_ref):
    h2 = jnp.maximum(
        jnp.dot(x_ref[...], w2_ref[...], preferred_element_type=jnp.float32),
        0.0)
    logits = jnp.dot(h2, wo_ref[...], preferred_element_type=jnp.float32)
    o_ref[...] = jax.nn.sigmoid(logits)


def _tc_head(s2, w2t, wot):
    return pl.pallas_call(
        _head_body,
        out_shape=jax.ShapeDtypeStruct((B, C), jnp.float32),
    )(s2, w2t, wot)


def kernel(nodes, features, adj, W1, W2, Wout):
    adjt = adj[:, :NS1].astype(jnp.int32).T.reshape(-1)
    w1t = (W1 * (1.0 / (NS1 + 1))).T
    w2t = (W2 * (1.0 / (NS2 + 1))).T
    wot = Wout.T

    y = _tc_matmul(features, w1t)
    nodes32 = nodes.astype(jnp.int32)
    h1, idx2 = _sc_layer1(y, adjt, nodes32)
    s2 = _sc_layer2(h1, nodes32, idx2)
    return _tc_head(s2, w2t, wot)
